# baseline (speedup 1.0000x reference)
import functools

import jax
import jax.numpy as jnp
from jax import lax
from jax.experimental import pallas as pl
from jax.experimental.pallas import tpu as pltpu
from jax.experimental.pallas import tpu_sc as plsc


@functools.lru_cache(maxsize=None)
def _build(batch, h_dim):
    info = plsc.get_sparse_core_info()
    nc, ns, nl = info.num_cores, info.num_subcores, info.num_lanes
    nw = nc * ns
    assert batch % (8 * nw) == 0 and h_dim % nl == 0
    b_per_w = batch // nw
    n_phases = 2
    b_per_ph = b_per_w // n_phases
    assert b_per_ph % nl == 0
    mesh = plsc.VectorSubcoreMesh(core_axis_name="c", subcore_axis_name="s")

    @functools.partial(
        pl.kernel,
        mesh=mesh,
        out_type=jax.ShapeDtypeStruct((batch, h_dim), jnp.float32),
        scratch_types=[
            pltpu.VMEM((b_per_ph, 1), jnp.int32),
            pltpu.VMEM((b_per_w, h_dim), jnp.float32),
            pltpu.SemaphoreType.DMA,
        ],
        compiler_params=pltpu.CompilerParams(needs_layout_passes=False),
    )
    def gather_kernel(idx_hbm, table_hbm, out_hbm, h_v, rows_v, sem):
        wid = lax.axis_index("s") * nc + lax.axis_index("c")
        base = wid * b_per_w
        zeros = jnp.zeros((nl,), jnp.int32)
        lane = lax.iota(jnp.int32, nl)

        for ph in range(n_phases):
            pltpu.sync_copy(
                idx_hbm.at[pl.ds(base + ph * b_per_ph, b_per_ph), :], h_v
            )

            def issue(g, _):
                v = plsc.load_gather(h_v, [g * nl + lane, zeros])
                for j in range(nl):
                    r = v[j]
                    pltpu.make_async_copy(
                        table_hbm.at[r],
                        rows_v.at[ph * b_per_ph + g * nl + j],
                        sem,
                    ).start()
                return 0

            lax.fori_loop(0, b_per_ph // nl, issue, 0)

        def drain(g, _):
            for j in range(nl):
                pltpu.make_async_copy(
                    table_hbm.at[0], rows_v.at[g * nl + j], sem
                ).wait()
            return 0

        lax.fori_loop(0, b_per_w // nl, drain, 0)
        pltpu.sync_copy(rows_v, out_hbm.at[pl.ds(base, b_per_w)])

    return gather_kernel


def kernel(g, h, embedding_table):
    idx2d = h.astype(jnp.int32)
    return _build(idx2d.shape[0], embedding_table.shape[1])(idx2d, embedding_table)


# speedup vs baseline: 2.1100x; 2.1100x over previous
import functools

import jax
import jax.numpy as jnp
from jax import lax
from jax.experimental import pallas as pl
from jax.experimental.pallas import tpu as pltpu
from jax.experimental.pallas import tpu_sc as plsc

_RING = 16


@functools.lru_cache(maxsize=None)
def _build(batch, h_dim):
    info = plsc.get_sparse_core_info()
    nc, ns, nl = info.num_cores, info.num_subcores, info.num_lanes
    nw = nc * ns
    assert batch % (8 * nw) == 0 and h_dim % nl == 0
    b_per_w = batch // nw
    half = b_per_w // 2
    n_groups = half // _RING
    mesh = plsc.VectorSubcoreMesh(core_axis_name="c", subcore_axis_name="s")

    scratch = [
        pltpu.VMEM((b_per_w,), jnp.int32),
        pltpu.VMEM((half, h_dim), jnp.float32),
    ]
    scratch += [pltpu.VMEM((h_dim, 128), jnp.float32) for _ in range(_RING)]
    scratch += [pltpu.SemaphoreType.DMA for _ in range(_RING)]

    @functools.partial(
        pl.kernel,
        mesh=mesh,
        out_type=jax.ShapeDtypeStruct((batch, h_dim), jnp.float32),
        scratch_types=scratch,
        compiler_params=pltpu.CompilerParams(needs_layout_passes=False),
    )
    def gather_kernel(idx_hbm, table_t_hbm, out_hbm, idx_v, rows_v, *ring):
        slots, sems = ring[:_RING], ring[_RING:]
        wid = lax.axis_index("s") * nc + lax.axis_index("c")
        base = wid * b_per_w
        iota = lax.iota(jnp.int32, nl)
        iota_hi = iota + nl

        pltpu.sync_copy(idx_hbm.at[pl.ds(base, b_per_w)], idx_v)

        for c in range(2):

            def group(gg, _, c=c):
                v = idx_v[pl.ds(c * half + gg * _RING, _RING)]
                lanes = []
                for j in range(_RING):
                    r = v[j]
                    tc_off = pl.multiple_of((r // 128) * 128, 128)
                    lanes.append(r % 128)
                    pltpu.make_async_copy(
                        table_t_hbm.at[:, pl.ds(tc_off, 128)],
                        slots[j],
                        sems[j],
                    ).start()
                for j in range(_RING):
                    pltpu.make_async_copy(
                        table_t_hbm.at[:, pl.ds(0, 128)], slots[j], sems[j]
                    ).wait()
                    splat = jnp.full((nl,), lanes[j], jnp.int32)
                    c0 = plsc.load_gather(slots[j], [iota, splat])
                    c1 = plsc.load_gather(slots[j], [iota_hi, splat])
                    rowv = jnp.full((nl,), gg * _RING + j, jnp.int32)
                    plsc.store_scatter(rows_v, [rowv, iota], c0)
                    plsc.store_scatter(rows_v, [rowv, iota_hi], c1)
                return 0

            lax.fori_loop(0, n_groups, group, 0)
            pltpu.sync_copy(
                rows_v, out_hbm.at[pl.ds(base + c * half, half)]
            )

    return gather_kernel


def kernel(g, h, embedding_table):
    idx = h.reshape(-1).astype(jnp.int32)
    return _build(idx.shape[0], embedding_table.shape[1])(
        idx, embedding_table.T
    )


# speedup vs baseline: 2.3880x; 1.1318x over previous
import functools

import jax
import jax.numpy as jnp
from jax import lax
from jax.experimental import pallas as pl
from jax.experimental.pallas import tpu as pltpu
from jax.experimental.pallas import tpu_sc as plsc

_RING = 16


@functools.lru_cache(maxsize=None)
def _build(batch, h_dim):
    info = plsc.get_sparse_core_info()
    nc, ns, nl = info.num_cores, info.num_subcores, info.num_lanes
    nw = nc * ns
    assert batch % (8 * nw) == 0 and h_dim % nl == 0
    b_per_w = batch // nw
    half = b_per_w // 2
    n_groups = half // _RING
    mesh = plsc.VectorSubcoreMesh(core_axis_name="c", subcore_axis_name="s")

    scratch = [
        pltpu.VMEM((b_per_w,), jnp.int32),
        pltpu.VMEM((half, h_dim), jnp.float32),
    ]
    scratch += [pltpu.VMEM((h_dim, 128), jnp.float32) for _ in range(_RING)]
    scratch += [pltpu.SemaphoreType.DMA for _ in range(_RING)]

    @functools.partial(
        pl.kernel,
        mesh=mesh,
        out_type=jax.ShapeDtypeStruct((batch, h_dim), jnp.float32),
        scratch_types=scratch,
        compiler_params=pltpu.CompilerParams(needs_layout_passes=False),
    )
    def gather_kernel(idx_hbm, table_t_hbm, out_hbm, idx_v, rows_v, *ring):
        slots, sems = ring[:_RING], ring[_RING:]
        wid = lax.axis_index("s") * nc + lax.axis_index("c")
        base = wid * b_per_w
        iota = lax.iota(jnp.int32, nl)
        iota_hi = iota + nl

        pltpu.sync_copy(idx_hbm.at[pl.ds(base, b_per_w)], idx_v)

        def issue(j, r):
            tc_off = pl.multiple_of((r // 128) * 128, 128)
            pltpu.make_async_copy(
                table_t_hbm.at[:, pl.ds(tc_off, 128)], slots[j], sems[j]
            ).start()

        def extract(j, lane, row):
            pltpu.make_async_copy(
                table_t_hbm.at[:, pl.ds(0, 128)], slots[j], sems[j]
            ).wait()
            splat = jnp.full((nl,), lane, jnp.int32)
            c0 = plsc.load_gather(slots[j], [iota, splat])
            c1 = plsc.load_gather(slots[j], [iota_hi, splat])
            rowv = jnp.full((nl,), row, jnp.int32)
            plsc.store_scatter(rows_v, [rowv, iota], c0)
            plsc.store_scatter(rows_v, [rowv, iota_hi], c1)

        for c in range(2):
            v0 = idx_v[pl.ds(c * half, _RING)]
            for j in range(_RING):
                issue(j, v0[j])
            lanes0 = tuple(v0[j] % 128 for j in range(_RING))

            def group(gg, lanes, c=c):
                vn = idx_v[pl.ds(c * half + (gg + 1) * _RING, _RING)]
                new_lanes = []
                for j in range(_RING):
                    extract(j, lanes[j], gg * _RING + j)
                    issue(j, vn[j])
                    new_lanes.append(vn[j] % 128)
                return tuple(new_lanes)

            lanes_last = lax.fori_loop(0, n_groups - 1, group, lanes0)
            for j in range(_RING):
                extract(j, lanes_last[j], (n_groups - 1) * _RING + j)
            pltpu.sync_copy(
                rows_v, out_hbm.at[pl.ds(base + c * half, half)]
            )

    return gather_kernel


def kernel(g, h, embedding_table):
    idx = h.reshape(-1).astype(jnp.int32)
    return _build(idx.shape[0], embedding_table.shape[1])(
        idx, embedding_table.T
    )


# speedup vs baseline: 2.5560x; 1.0703x over previous
import functools

import jax
import jax.numpy as jnp
from jax import lax
from jax.experimental import pallas as pl
from jax.experimental.pallas import tpu as pltpu
from jax.experimental.pallas import tpu_sc as plsc

_RING = 16


@functools.lru_cache(maxsize=None)
def _build(batch, h_dim):
    info = plsc.get_sparse_core_info()
    nc, ns, nl = info.num_cores, info.num_subcores, info.num_lanes
    nw = nc * ns
    assert batch % (128 * nw) == 0 and h_dim % nl == 0
    b_per_w = batch // nw
    n_groups = b_per_w // _RING
    mesh = plsc.VectorSubcoreMesh(core_axis_name="c", subcore_axis_name="s")

    scratch = [
        pltpu.VMEM((b_per_w,), jnp.int32),
        pltpu.VMEM((h_dim, b_per_w), jnp.float32),
    ]
    scratch += [pltpu.VMEM((h_dim, 128), jnp.float32) for _ in range(_RING)]
    scratch += [pltpu.SemaphoreType.DMA for _ in range(_RING)]

    @functools.partial(
        pl.kernel,
        mesh=mesh,
        out_type=jax.ShapeDtypeStruct((h_dim, batch), jnp.float32),
        scratch_types=scratch,
        compiler_params=pltpu.CompilerParams(needs_layout_passes=False),
    )
    def gather_kernel(idx_hbm, table_t_hbm, out_t_hbm, idx_v, stage_v, *ring):
        slots, sems = ring[:_RING], ring[_RING:]
        wid = lax.axis_index("s") * nc + lax.axis_index("c")
        base = pl.multiple_of(wid * b_per_w, 128)
        iota = lax.iota(jnp.int32, nl)
        iota_hi = iota + nl

        pltpu.sync_copy(idx_hbm.at[pl.ds(base, b_per_w)], idx_v)

        def issue(j, r):
            tc_off = pl.multiple_of((r // 128) * 128, 128)
            pltpu.make_async_copy(
                table_t_hbm.at[:, pl.ds(tc_off, 128)], slots[j], sems[j]
            ).start()

        def extract(j, lane, col):
            pltpu.make_async_copy(
                table_t_hbm.at[:, pl.ds(0, 128)], slots[j], sems[j]
            ).wait()
            splat = jnp.full((nl,), lane, jnp.int32)
            c0 = plsc.load_gather(slots[j], [iota, splat])
            c1 = plsc.load_gather(slots[j], [iota_hi, splat])
            colv = jnp.full((nl,), col, jnp.int32)
            plsc.store_scatter(stage_v, [iota, colv], c0)
            plsc.store_scatter(stage_v, [iota_hi, colv], c1)

        v0 = idx_v[pl.ds(0, _RING)]
        for j in range(_RING):
            issue(j, v0[j])
        lanes0 = tuple(v0[j] % 128 for j in range(_RING))

        def group(gg, lanes):
            vn = idx_v[pl.ds((gg + 1) * _RING, _RING)]
            new_lanes = []
            for j in range(_RING):
                extract(j, lanes[j], gg * _RING + j)
                issue(j, vn[j])
                new_lanes.append(vn[j] % 128)
            return tuple(new_lanes)

        lanes_last = lax.fori_loop(0, n_groups - 1, group, lanes0)
        for j in range(_RING):
            extract(j, lanes_last[j], (n_groups - 1) * _RING + j)

        pltpu.sync_copy(stage_v, out_t_hbm.at[:, pl.ds(base, b_per_w)])

    return gather_kernel


def kernel(g, h, embedding_table):
    idx = h.reshape(-1).astype(jnp.int32)
    out_t = _build(idx.shape[0], embedding_table.shape[1])(
        idx, embedding_table.T
    )
    return out_t.T
